# Initial kernel scaffold; baseline (speedup 1.0000x reference)
import jax
import jax.numpy as jnp
from jax.experimental import pallas as pl


def kernel(events):
    raise NotImplementedError("write your pallas kernel here")


# speedup vs baseline: 4.2819x; 4.2819x over previous
import functools

import jax
import jax.numpy as jnp
from jax import lax
from jax.experimental import pallas as pl
from jax.experimental.pallas import tpu as pltpu
from jax.experimental.pallas import tpu_sc as plsc

H = 260
W = 346
S = 16
NEV = 1000000
BN = 4
NPB = NEV // BN
CE = 2000
NCH = NPB // CE
HWPAD = 89968
HH = H // 2
WH = W // 2
NHALF = HH * WH
NHPAD = 22496
BIG = 4 * NHPAD
NUNITS = BN * S + BN
NTILES = 32


def _sc_body(x_hbm, y_hbm, t_hbm, tmax_hbm, dil_hbm, stats_hbm,
             big, evx, evy, evt, tmv):
    pltpu.sync_copy(tmax_hbm, tmv)
    wid = lax.axis_index("s") * 2 + lax.axis_index("c")
    base_iota = lax.iota(jnp.int32, 16)
    zeros_i = jnp.zeros((16,), jnp.int32)
    ones_f = jnp.ones((16,), jnp.float32)

    def process_unit(u):
        def zbody(i, carry):
            big[pl.ds(i * 16, 16)] = jnp.zeros((16,), jnp.float32)
            return carry

        lax.fori_loop(0, BIG // 16, zbody, 0)

        is_dil = u < BN * S
        b = jnp.where(is_dil, u // S, u - BN * S)
        s = u % S
        tmax_splat = plsc.load_gather(tmv, [jnp.broadcast_to(b, (16,))])
        s_splat = zeros_i + s

        def chunk_body(c, carry):
            start = b * NPB + c * CE
            pltpu.sync_copy(x_hbm.at[pl.ds(start, CE)], evx)
            pltpu.sync_copy(y_hbm.at[pl.ds(start, CE)], evy)
            pltpu.sync_copy(t_hbm.at[pl.ds(start, CE)], evt)

            def grp(g, carry2):
                x = evx[pl.ds(g * 16, 16)]
                y = evy[pl.ds(g * 16, 16)]
                t = evt[pl.ds(g * 16, 16)]
                xi = x.astype(jnp.int32)
                yi = y.astype(jnp.int32)
                tn = t / tmax_splat
                ts = jnp.minimum((tn * float(S)).astype(jnp.int32), S - 1)

                @pl.when(is_dil)
                def _():
                    pidx = yi * W + xi
                    plsc.store_scatter(big, [pidx], ones_f,
                                       mask=(ts == s_splat))

                @pl.when(jnp.logical_not(is_dil))
                def _():
                    hidx = (yi >> 1) * WH + (xi >> 1)
                    sx = (1 - ((xi & 1) << 1)).astype(jnp.float32)
                    sy = (1 - ((yi & 1) << 1)).astype(jnp.float32)
                    plsc.addupdate_scatter(big, [hidx], sx)
                    plsc.addupdate_scatter(big, [hidx + NHPAD], sy)
                    plsc.addupdate_scatter(big, [hidx + 2 * NHPAD], tn)
                    plsc.addupdate_scatter(big, [hidx + 3 * NHPAD], ones_f)

                return carry2

            lax.fori_loop(0, CE // 16, grp, 0)
            return carry

        lax.fori_loop(0, NCH, chunk_body, 0)

        @pl.when(is_dil)
        def _():
            pltpu.sync_copy(big, dil_hbm.at[u])

        @pl.when(jnp.logical_not(is_dil))
        def _():
            pltpu.sync_copy(big, stats_hbm.at[u - BN * S])

    for k in range((NUNITS + NTILES - 1) // NTILES):
        u = wid + NTILES * k

        @pl.when(u < NUNITS)
        def _():
            process_unit(u)


_sc_scatter = functools.partial(
    pl.kernel,
    out_type=(
        jax.ShapeDtypeStruct((BN * S, BIG), jnp.float32),
        jax.ShapeDtypeStruct((BN, BIG), jnp.float32),
    ),
    mesh=plsc.VectorSubcoreMesh(core_axis_name="c", subcore_axis_name="s"),
    scratch_types=[
        pltpu.VMEM((BIG,), jnp.float32),
        pltpu.VMEM((CE,), jnp.float32),
        pltpu.VMEM((CE,), jnp.float32),
        pltpu.VMEM((CE,), jnp.float32),
        pltpu.VMEM((16,), jnp.float32),
    ],
    compiler_params=pltpu.CompilerParams(needs_layout_passes=False),
)(_sc_body)


def _tmax_body(t_ref, o_ref):
    o_ref[...] = jnp.max(t_ref[...], axis=1, keepdims=True)


def _erode_body(d_ref, o_ref, pad_ref):
    pad_ref[...] = jnp.zeros((H + 2, W + 2), jnp.float32)
    prev = d_ref[0, 0]
    o_ref[0, 0] = prev
    for i in range(1, S):
        cur = d_ref[0, i]
        mix = cur * (0.5 + (S - i) / S) + prev * (i / S)
        pad_ref[1:H + 1, 1:W + 1] = mix
        nsum = None
        for dy_off in range(3):
            for dx_off in range(3):
                if dy_off == 1 and dx_off == 1:
                    continue
                sl = pad_ref[dy_off:dy_off + H, dx_off:dx_off + W]
                nsum = sl if nsum is None else nsum + sl
        new = nsum * 0.125 - 0.25
        o_ref[0, i] = new
        prev = new


def kernel(events):
    t2 = events[:, 2].reshape(BN, NPB)
    tmax = pl.pallas_call(
        _tmax_body,
        out_shape=jax.ShapeDtypeStruct((BN, 1), jnp.float32),
    )(t2)
    tmax16 = jnp.concatenate([tmax[:, 0], jnp.ones((16 - BN,), jnp.float32)])

    dil_rows, stats_rows = _sc_scatter(
        events[:, 0], events[:, 1], events[:, 2], tmax16)

    dil = dil_rows[:, :H * W].reshape(BN, S, H, W)
    stats = stats_rows.reshape(BN, 4, NHPAD)[:, :, :NHALF]
    stats = stats.reshape(BN, 4, HH, WH)

    dil2 = pl.pallas_call(
        _erode_body,
        grid=(BN,),
        in_specs=[pl.BlockSpec((1, S, H, W), lambda b: (b, 0, 0, 0))],
        out_specs=pl.BlockSpec((1, S, H, W), lambda b: (b, 0, 0, 0)),
        out_shape=jax.ShapeDtypeStruct((BN, S, H, W), jnp.float32),
        scratch_shapes=[pltpu.VMEM((H + 2, W + 2), jnp.float32)],
    )(dil)

    counts = jnp.sum(dil2 <= 0, axis=(2, 3))
    best = jnp.argmax(counts, axis=1)
    best_dil = jnp.take_along_axis(
        dil2, best[:, None, None, None], axis=1)[:, 0][:, ::2, ::2]

    dx = stats[:, 0]
    dy = stats[:, 1]
    counter = stats[:, 3]
    divider = jnp.where(counter == 0, 1.0, counter)
    timer = stats[:, 2] / divider
    return jnp.stack([dx, dy, timer, counter, best_dil], axis=1)


# speedup vs baseline: 5.9947x; 1.4000x over previous
import functools

import jax
import jax.numpy as jnp
from jax import lax
from jax.experimental import pallas as pl
from jax.experimental.pallas import tpu as pltpu
from jax.experimental.pallas import tpu_sc as plsc

H = 260
W = 346
S = 16
NEV = 1000000
BN = 4
NPB = NEV // BN
CE = 10000
NCH = NPB // CE
HWPAD = 89968
HH = H // 2
WH = W // 2
NHALF = HH * WH
NHPAD = 22496
BIG = 4 * NHPAD
NUNITS = BN * S + BN
NTILES = 32


def _sc_body(x_hbm, y_hbm, t_hbm, tmax_hbm, dil_hbm, stats_hbm,
             big, evx, evy, evt, tmv):
    pltpu.sync_copy(tmax_hbm, tmv)
    wid = lax.axis_index("s") * 2 + lax.axis_index("c")
    base_iota = lax.iota(jnp.int32, 16)
    zeros_i = jnp.zeros((16,), jnp.int32)
    ones_f = jnp.ones((16,), jnp.float32)

    def process_unit(u):
        def zbody(i, carry):
            big[pl.ds(i * 16, 16)] = jnp.zeros((16,), jnp.float32)
            return carry

        lax.fori_loop(0, BIG // 16, zbody, 0)

        is_dil = u < BN * S
        b = jnp.where(is_dil, u // S, u - BN * S)
        s = u % S
        tmax_splat = plsc.load_gather(tmv, [jnp.broadcast_to(b, (16,))])
        s_splat = zeros_i + s

        def chunk_body(c, carry):
            start = b * NPB + c * CE
            pltpu.sync_copy(x_hbm.at[pl.ds(start, CE)], evx)
            pltpu.sync_copy(y_hbm.at[pl.ds(start, CE)], evy)
            pltpu.sync_copy(t_hbm.at[pl.ds(start, CE)], evt)

            def grp(g, carry2):
                x = evx[pl.ds(g * 16, 16)]
                y = evy[pl.ds(g * 16, 16)]
                t = evt[pl.ds(g * 16, 16)]
                xi = x.astype(jnp.int32)
                yi = y.astype(jnp.int32)
                tn = t / tmax_splat
                ts = jnp.minimum((tn * float(S)).astype(jnp.int32), S - 1)

                @pl.when(is_dil)
                def _():
                    pidx = yi * W + xi
                    plsc.store_scatter(big, [pidx], ones_f,
                                       mask=(ts == s_splat))

                @pl.when(jnp.logical_not(is_dil))
                def _():
                    hidx = (yi >> 1) * WH + (xi >> 1)
                    sx = (1 - ((xi & 1) << 1)).astype(jnp.float32)
                    sy = (1 - ((yi & 1) << 1)).astype(jnp.float32)
                    plsc.addupdate_scatter(big, [hidx], sx)
                    plsc.addupdate_scatter(big, [hidx + NHPAD], sy)
                    plsc.addupdate_scatter(big, [hidx + 2 * NHPAD], tn)
                    plsc.addupdate_scatter(big, [hidx + 3 * NHPAD], ones_f)

                return carry2

            lax.fori_loop(0, CE // 16, grp, 0)
            return carry

        lax.fori_loop(0, NCH, chunk_body, 0)

        @pl.when(is_dil)
        def _():
            pltpu.sync_copy(big, dil_hbm.at[u])

        @pl.when(jnp.logical_not(is_dil))
        def _():
            pltpu.sync_copy(big, stats_hbm.at[u - BN * S])

    for k in range((NUNITS + NTILES - 1) // NTILES):
        u = wid + NTILES * k

        @pl.when(u < NUNITS)
        def _():
            process_unit(u)


_sc_scatter = functools.partial(
    pl.kernel,
    out_type=(
        jax.ShapeDtypeStruct((BN * S, BIG), jnp.float32),
        jax.ShapeDtypeStruct((BN, BIG), jnp.float32),
    ),
    mesh=plsc.VectorSubcoreMesh(core_axis_name="c", subcore_axis_name="s"),
    scratch_types=[
        pltpu.VMEM((BIG,), jnp.float32),
        pltpu.VMEM((CE,), jnp.float32),
        pltpu.VMEM((CE,), jnp.float32),
        pltpu.VMEM((CE,), jnp.float32),
        pltpu.VMEM((16,), jnp.float32),
    ],
    compiler_params=pltpu.CompilerParams(needs_layout_passes=False),
)(_sc_body)


def _tmax_body(t_ref, o_ref):
    o_ref[...] = jnp.max(t_ref[...], axis=1, keepdims=True)


def _erode_body(d_ref, o_ref, pad_ref):
    pad_ref[...] = jnp.zeros((H + 2, W + 2), jnp.float32)
    prev = d_ref[0, 0]
    o_ref[0, 0] = prev
    for i in range(1, S):
        cur = d_ref[0, i]
        mix = cur * (0.5 + (S - i) / S) + prev * (i / S)
        pad_ref[1:H + 1, 1:W + 1] = mix
        nsum = None
        for dy_off in range(3):
            for dx_off in range(3):
                if dy_off == 1 and dx_off == 1:
                    continue
                sl = pad_ref[dy_off:dy_off + H, dx_off:dx_off + W]
                nsum = sl if nsum is None else nsum + sl
        new = nsum * 0.125 - 0.25
        o_ref[0, i] = new
        prev = new


def kernel(events):
    t2 = events[:, 2].reshape(BN, NPB)
    tmax = pl.pallas_call(
        _tmax_body,
        out_shape=jax.ShapeDtypeStruct((BN, 1), jnp.float32),
    )(t2)
    tmax16 = jnp.concatenate([tmax[:, 0], jnp.ones((16 - BN,), jnp.float32)])

    dil_rows, stats_rows = _sc_scatter(
        events[:, 0], events[:, 1], events[:, 2], tmax16)

    dil = dil_rows[:, :H * W].reshape(BN, S, H, W)
    stats = stats_rows.reshape(BN, 4, NHPAD)[:, :, :NHALF]
    stats = stats.reshape(BN, 4, HH, WH)

    dil2 = pl.pallas_call(
        _erode_body,
        grid=(BN,),
        in_specs=[pl.BlockSpec((1, S, H, W), lambda b: (b, 0, 0, 0))],
        out_specs=pl.BlockSpec((1, S, H, W), lambda b: (b, 0, 0, 0)),
        out_shape=jax.ShapeDtypeStruct((BN, S, H, W), jnp.float32),
        scratch_shapes=[pltpu.VMEM((H + 2, W + 2), jnp.float32)],
    )(dil)

    counts = jnp.sum(dil2 <= 0, axis=(2, 3))
    best = jnp.argmax(counts, axis=1)
    best_dil = jnp.take_along_axis(
        dil2, best[:, None, None, None], axis=1)[:, 0][:, ::2, ::2]

    dx = stats[:, 0]
    dy = stats[:, 1]
    counter = stats[:, 3]
    divider = jnp.where(counter == 0, 1.0, counter)
    timer = stats[:, 2] / divider
    return jnp.stack([dx, dy, timer, counter, best_dil], axis=1)
